# baseline (speedup 1.0000x reference)
import functools

import jax
import jax.numpy as jnp
from jax import lax
from jax.experimental import pallas as pl
from jax.experimental.pallas import tpu as pltpu
from jax.experimental.pallas import tpu_sc as plsc

B = 8
N = 4096
NCHUNK = 4
CH = N // NCHUNK
L = 16
JV = 4
INF = float("inf")


def _sc_chamfer(go_t, gq_t, po_t, pq2_t):
    mesh = plsc.VectorSubcoreMesh(core_axis_name="c", subcore_axis_name="s")

    @functools.partial(
        pl.kernel,
        mesh=mesh,
        out_type=[
            jax.ShapeDtypeStruct((B, NCHUNK, N), jnp.float32),
            jax.ShapeDtypeStruct((B, N), jnp.float32),
        ],
        scratch_types=[
            pltpu.VMEM((3, N), jnp.float32),
            pltpu.VMEM((3, N), jnp.float32),
            pltpu.VMEM((3, CH), jnp.float32),
            pltpu.VMEM((3, CH), jnp.float32),
            pltpu.VMEM((N,), jnp.float32),
            pltpu.VMEM((CH,), jnp.float32),
            pltpu.VMEM((CH,), jnp.float32),
            pltpu.VMEM((N,), jnp.float32),
        ],
    )
    def body(go_hbm, gq_hbm, po_hbm, pq2_hbm, rowpart_hbm, colmin_hbm,
             go_v, gq_v, po_v, pq2_v, gn_v, pn_v, cmin_v, rpart_v):
        c = lax.axis_index("c")
        s = lax.axis_index("s")
        wid = c * 16 + s
        b = wid // NCHUNK
        ch = wid % NCHUNK

        pltpu.sync_copy(go_hbm.at[b], go_v)
        pltpu.sync_copy(gq_hbm.at[b], gq_v)
        psl = pl.ds(ch * CH, CH)
        pltpu.sync_copy(po_hbm.at[b, :, psl], po_v)
        pltpu.sync_copy(pq2_hbm.at[b, :, psl], pq2_v)

        ridx = (lax.iota(jnp.int32, L) + 1) & (L - 1)

        def rot1(v):
            return jnp.take_along_axis(v, ridx, axis=0)

        def init_gn(iv, _):
            sl = pl.ds(iv * L, L)
            x = go_v[0, sl]
            y = go_v[1, sl]
            z = go_v[2, sl]
            gn_v[sl] = x * x + y * y + z * z
            return 0

        lax.fori_loop(0, N // L, init_gn, 0)

        def init_pn(jv, _):
            sl = pl.ds(jv * L, L)
            x = po_v[0, sl]
            y = po_v[1, sl]
            z = po_v[2, sl]
            pn_v[sl] = x * x + y * y + z * z
            cmin_v[sl] = jnp.full((L,), INF, jnp.float32)
            return 0

        lax.fori_loop(0, CH // L, init_pn, 0)

        def body_ib(ib, _):
            gsl = pl.ds(ib * L, L)
            gx0 = gq_v[0, gsl]
            gy0 = gq_v[1, gsl]
            gz0 = gq_v[2, gsl]
            gn0 = gn_v[gsl]

            def body_jq(jq, rmin):
                sls = [pl.ds((jq * JV + t) * L, L) for t in range(JV)]
                px = [pq2_v[0, sl] for sl in sls]
                py = [pq2_v[1, sl] for sl in sls]
                pz = [pq2_v[2, sl] for sl in sls]
                pn = [pn_v[sl] for sl in sls]
                cm = [cmin_v[sl] for sl in sls]
                gx, gy, gz, gn, rm = gx0, gy0, gz0, gn0, rmin
                for k in range(L):
                    if k > 0:
                        gx = rot1(gx)
                        gy = rot1(gy)
                        gz = rot1(gz)
                        gn = rot1(gn)
                        rm = rot1(rm)
                    for t in range(JV):
                        t3 = gx * px[t] + gy * py[t] + gz * pz[t]
                        dist = t3 + (gn + pn[t])
                        cm[t] = jnp.minimum(cm[t], dist)
                        rm = jnp.minimum(rm, dist)
                rm = rot1(rm)
                for t in range(JV):
                    cmin_v[sls[t]] = cm[t]
                return rm

            rmin = lax.fori_loop(
                0, CH // (L * JV), body_jq, jnp.full((L,), INF, jnp.float32)
            )
            rpart_v[gsl] = rmin
            return 0

        lax.fori_loop(0, N // L, body_ib, 0)

        pltpu.sync_copy(rpart_v, rowpart_hbm.at[b, ch])
        pltpu.sync_copy(cmin_v, colmin_hbm.at[b, psl])

    return body(go_t, gq_t, po_t, pq2_t)


def _combine_tc(rowpart, colmin):
    def body(rp_ref, cm_ref, l1_ref, l2_ref):
        cm = cm_ref[...]
        rp = rp_ref[...]
        l1 = jnp.mean(cm, axis=-1)
        l2 = jnp.mean(jnp.min(rp, axis=1), axis=-1)
        l1_ref[...] = jnp.broadcast_to(l1[:, None], (B, 128))
        l2_ref[...] = jnp.broadcast_to(l2[:, None], (B, 128))

    out = pl.pallas_call(
        body,
        out_shape=[
            jax.ShapeDtypeStruct((B, 128), jnp.float32),
            jax.ShapeDtypeStruct((B, 128), jnp.float32),
        ],
    )(rowpart, colmin)
    return out[0][:, 0], out[1][:, 0]


def kernel(preds, gts):
    go_t = jnp.transpose(gts, (0, 2, 1))
    po_t = jnp.transpose(preds, (0, 2, 1))
    gq16, pq16 = lax.optimization_barrier(
        (go_t.astype(jnp.bfloat16), po_t.astype(jnp.bfloat16))
    )
    gq_t = gq16.astype(jnp.float32)
    pq2_t = -2.0 * pq16.astype(jnp.float32)
    rowpart, colmin = _sc_chamfer(go_t, gq_t, po_t, pq2_t)
    loss_1, loss_2 = _combine_tc(rowpart, colmin)
    return (loss_1, loss_2)


# speedup vs baseline: 3.5574x; 3.5574x over previous
import functools

import jax
import jax.numpy as jnp
from jax import lax
from jax.experimental import pallas as pl
from jax.experimental.pallas import tpu as pltpu
from jax.experimental.pallas import tpu_sc as plsc

B = 8
N = 4096
NPS = 512
NPT = N - NPS
NCHUNK = 4
CHS = NPS // NCHUNK
L = 16
JV = 4
TI = 512
INF = float("inf")


def _sc_chamfer(go_t, gq_t, po_t, pq2_t):
    mesh = plsc.VectorSubcoreMesh(core_axis_name="c", subcore_axis_name="s")

    @functools.partial(
        pl.kernel,
        mesh=mesh,
        out_type=[
            jax.ShapeDtypeStruct((B, NCHUNK, N), jnp.float32),
            jax.ShapeDtypeStruct((B, NPS), jnp.float32),
        ],
        scratch_types=[
            pltpu.VMEM((3, N), jnp.float32),
            pltpu.VMEM((3, N), jnp.float32),
            pltpu.VMEM((3, CHS), jnp.float32),
            pltpu.VMEM((3, CHS), jnp.float32),
            pltpu.VMEM((N,), jnp.float32),
            pltpu.VMEM((CHS,), jnp.float32),
            pltpu.VMEM((CHS,), jnp.float32),
            pltpu.VMEM((N,), jnp.float32),
        ],
    )
    def body(go_hbm, gq_hbm, po_hbm, pq2_hbm, rowpart_hbm, colmin_hbm,
             go_v, gq_v, po_v, pq2_v, gn_v, pn_v, cmin_v, rpart_v):
        c = lax.axis_index("c")
        s = lax.axis_index("s")
        wid = c * 16 + s
        b = wid // NCHUNK
        ch = wid % NCHUNK

        pltpu.sync_copy(go_hbm.at[b], go_v)
        pltpu.sync_copy(gq_hbm.at[b], gq_v)
        psl = pl.ds(ch * CHS, CHS)
        pltpu.sync_copy(po_hbm.at[b, :, psl], po_v)
        pltpu.sync_copy(pq2_hbm.at[b, :, psl], pq2_v)

        ridx = (lax.iota(jnp.int32, L) + 1) & (L - 1)

        def rot1(v):
            return jnp.take_along_axis(v, ridx, axis=0)

        def init_gn(iv, _):
            sl = pl.ds(iv * L, L)
            x = go_v[0, sl]
            y = go_v[1, sl]
            z = go_v[2, sl]
            gn_v[sl] = x * x + y * y + z * z
            return 0

        lax.fori_loop(0, N // L, init_gn, 0)

        def init_pn(jv, _):
            sl = pl.ds(jv * L, L)
            x = po_v[0, sl]
            y = po_v[1, sl]
            z = po_v[2, sl]
            pn_v[sl] = x * x + y * y + z * z
            cmin_v[sl] = jnp.full((L,), INF, jnp.float32)
            return 0

        lax.fori_loop(0, CHS // L, init_pn, 0)

        def body_ib(ib, _):
            gsl = pl.ds(ib * L, L)
            gx0 = gq_v[0, gsl]
            gy0 = gq_v[1, gsl]
            gz0 = gq_v[2, gsl]
            gn0 = gn_v[gsl]

            def body_jq(jq, rmin):
                sls = [pl.ds((jq * JV + t) * L, L) for t in range(JV)]
                px = [pq2_v[0, sl] for sl in sls]
                py = [pq2_v[1, sl] for sl in sls]
                pz = [pq2_v[2, sl] for sl in sls]
                pn = [pn_v[sl] for sl in sls]
                cm = [cmin_v[sl] for sl in sls]
                gx, gy, gz, gn, rm = gx0, gy0, gz0, gn0, rmin
                for k in range(L):
                    if k > 0:
                        gx = rot1(gx)
                        gy = rot1(gy)
                        gz = rot1(gz)
                        gn = rot1(gn)
                        rm = rot1(rm)
                    for t in range(JV):
                        t3 = gx * px[t] + gy * py[t] + gz * pz[t]
                        dist = t3 + (gn + pn[t])
                        cm[t] = jnp.minimum(cm[t], dist)
                        rm = jnp.minimum(rm, dist)
                rm = rot1(rm)
                for t in range(JV):
                    cmin_v[sls[t]] = cm[t]
                return rm

            rmin = lax.fori_loop(
                0, CHS // (L * JV), body_jq, jnp.full((L,), INF, jnp.float32)
            )
            rpart_v[gsl] = rmin
            return 0

        lax.fori_loop(0, N // L, body_ib, 0)

        pltpu.sync_copy(rpart_v, rowpart_hbm.at[b, ch])
        pltpu.sync_copy(cmin_v, colmin_hbm.at[b, psl])

    return body(go_t, gq_t, po_t, pq2_t)


def _tc_chamfer(gts, po_t, gq8, pq2t):
    def body(go_ref, po_ref, gq8_ref, pq2_ref, rp_ref, cm_ref):
        it = pl.program_id(1)
        zz2 = jnp.dot(gq8_ref[0], pq2_ref[0],
                      preferred_element_type=jnp.float32)
        po = po_ref[0]
        pn = po[0] * po[0] + po[1] * po[1] + po[2] * po[2]
        go = go_ref[0]
        gn = jnp.sum(go * go, axis=1)
        P = zz2 + (gn[:, None] + pn[None, :])
        rp_ref[0, 0, pl.ds(it * TI, TI)] = jnp.min(P, axis=1)
        ctile = jnp.min(P, axis=0)

        @pl.when(it == 0)
        def _():
            cm_ref[0, 0] = ctile

        @pl.when(it > 0)
        def _():
            cm_ref[0, 0] = jnp.minimum(cm_ref[0, 0], ctile)

    grid = (B, N // TI)
    rowpart, colmin = pl.pallas_call(
        body,
        grid=grid,
        in_specs=[
            pl.BlockSpec((1, TI, 3), lambda b, i: (b, i, 0)),
            pl.BlockSpec((1, 3, NPT), lambda b, i: (b, 0, 0)),
            pl.BlockSpec((1, TI, 8), lambda b, i: (b, i, 0)),
            pl.BlockSpec((1, 8, NPT), lambda b, i: (b, 0, 0)),
        ],
        out_specs=[
            pl.BlockSpec((1, 1, N), lambda b, i: (b, 0, 0)),
            pl.BlockSpec((1, 1, NPT), lambda b, i: (b, 0, 0)),
        ],
        out_shape=[
            jax.ShapeDtypeStruct((B, 1, N), jnp.float32),
            jax.ShapeDtypeStruct((B, 1, NPT), jnp.float32),
        ],
    )(gts, po_t, gq8, pq2t)
    return rowpart, colmin[:, 0, :]


def _combine_tc(rp_stack, cm_all):
    M = rp_stack.shape[1]

    def body(rp_ref, cm_ref, l1_ref, l2_ref):
        cm = cm_ref[...]
        rp = rp_ref[...]
        l1 = jnp.mean(cm, axis=-1)
        l2 = jnp.mean(jnp.min(rp, axis=1), axis=-1)
        l1_ref[...] = jnp.broadcast_to(l1[:, None], (B, 128))
        l2_ref[...] = jnp.broadcast_to(l2[:, None], (B, 128))

    out = pl.pallas_call(
        body,
        out_shape=[
            jax.ShapeDtypeStruct((B, 128), jnp.float32),
            jax.ShapeDtypeStruct((B, 128), jnp.float32),
        ],
    )(rp_stack, cm_all)
    return out[0][:, 0], out[1][:, 0]


def kernel(preds, gts):
    gq16, pq16 = lax.optimization_barrier(
        (gts.astype(jnp.bfloat16), preds.astype(jnp.bfloat16))
    )
    zeros_g = jnp.zeros((B, N, 5), jnp.bfloat16)
    gq8 = jnp.concatenate([gq16, zeros_g], axis=-1)
    pq2 = (-2.0 * pq16.astype(jnp.float32)).astype(jnp.bfloat16)
    pq2t_tc = jnp.concatenate(
        [pq2[:, NPS:, :], jnp.zeros((B, NPT, 5), jnp.bfloat16)], axis=-1
    ).transpose(0, 2, 1)
    po_t = jnp.transpose(preds, (0, 2, 1))

    go_t = jnp.transpose(gts, (0, 2, 1))
    gq_t = jnp.transpose(gq16, (0, 2, 1)).astype(jnp.float32)
    po_sc = po_t[:, :, :NPS]
    pq2_sc = jnp.transpose(pq2[:, :NPS, :], (0, 2, 1)).astype(jnp.float32)
    rowpart_sc, colmin_sc = _sc_chamfer(go_t, gq_t, po_sc, pq2_sc)

    rowpart_tc, colmin_tc = _tc_chamfer(gts, po_t[:, :, NPS:], gq8, pq2t_tc)

    rp_stack = jnp.concatenate([rowpart_tc, rowpart_sc], axis=1)
    cm_all = jnp.concatenate([colmin_sc, colmin_tc], axis=1)
    loss_1, loss_2 = _combine_tc(rp_stack, cm_all)
    return (loss_1, loss_2)


# speedup vs baseline: 3.8296x; 1.0765x over previous
import functools

import jax
import jax.numpy as jnp
from jax import lax
from jax.experimental import pallas as pl
from jax.experimental.pallas import tpu as pltpu
from jax.experimental.pallas import tpu_sc as plsc

B = 8
N = 4096
NPS = 512
NPT = N - NPS
NCHUNK = 4
CHS = NPS // NCHUNK
L = 16
JV = 4
TI = 512
INF = float("inf")


def _sc_chamfer(go_t, gq_t, po_t, pq2_t):
    mesh = plsc.VectorSubcoreMesh(core_axis_name="c", subcore_axis_name="s")

    @functools.partial(
        pl.kernel,
        mesh=mesh,
        out_type=[
            jax.ShapeDtypeStruct((B, NCHUNK, N), jnp.float32),
            jax.ShapeDtypeStruct((B, NPS), jnp.float32),
        ],
        scratch_types=[
            pltpu.VMEM((3, N), jnp.float32),
            pltpu.VMEM((3, N), jnp.float32),
            pltpu.VMEM((3, CHS), jnp.float32),
            pltpu.VMEM((3, CHS), jnp.float32),
            pltpu.VMEM((N,), jnp.float32),
            pltpu.VMEM((CHS,), jnp.float32),
            pltpu.VMEM((CHS,), jnp.float32),
            pltpu.VMEM((N,), jnp.float32),
        ],
    )
    def body(go_hbm, gq_hbm, po_hbm, pq2_hbm, rowpart_hbm, colmin_hbm,
             go_v, gq_v, po_v, pq2_v, gn_v, pn_v, cmin_v, rpart_v):
        c = lax.axis_index("c")
        s = lax.axis_index("s")
        wid = c * 16 + s
        b = wid // NCHUNK
        ch = wid % NCHUNK

        pltpu.sync_copy(go_hbm.at[b], go_v)
        pltpu.sync_copy(gq_hbm.at[b], gq_v)
        psl = pl.ds(ch * CHS, CHS)
        pltpu.sync_copy(po_hbm.at[b, :, psl], po_v)
        pltpu.sync_copy(pq2_hbm.at[b, :, psl], pq2_v)

        ridx = (lax.iota(jnp.int32, L) + 1) & (L - 1)

        def rot1(v):
            return jnp.take_along_axis(v, ridx, axis=0)

        def init_gn(iv, _):
            sl = pl.ds(iv * L, L)
            x = go_v[0, sl]
            y = go_v[1, sl]
            z = go_v[2, sl]
            gn_v[sl] = x * x + y * y + z * z
            return 0

        lax.fori_loop(0, N // L, init_gn, 0)

        def init_pn(jv, _):
            sl = pl.ds(jv * L, L)
            x = po_v[0, sl]
            y = po_v[1, sl]
            z = po_v[2, sl]
            pn_v[sl] = x * x + y * y + z * z
            cmin_v[sl] = jnp.full((L,), INF, jnp.float32)
            return 0

        lax.fori_loop(0, CHS // L, init_pn, 0)

        def body_ib(ib, _):
            gsl = pl.ds(ib * L, L)
            gx0 = gq_v[0, gsl]
            gy0 = gq_v[1, gsl]
            gz0 = gq_v[2, gsl]
            gn0 = gn_v[gsl]

            def body_jq(jq, rmin):
                sls = [pl.ds((jq * JV + t) * L, L) for t in range(JV)]
                px = [pq2_v[0, sl] for sl in sls]
                py = [pq2_v[1, sl] for sl in sls]
                pz = [pq2_v[2, sl] for sl in sls]
                cm = [cmin_v[sl] for sl in sls]
                pn = [pn_v[sl] for sl in sls]
                gx, gy, gz, gn, rm = gx0, gy0, gz0, gn0, rmin
                for k in range(L):
                    if k > 0:
                        gx = rot1(gx)
                        gy = rot1(gy)
                        gz = rot1(gz)
                        gn = rot1(gn)
                        rm = rot1(rm)
                    d = []
                    for t in range(JV):
                        t3 = gx * px[t] + gy * py[t] + gz * pz[t]
                        dist = t3 + (gn + pn[t])
                        cm[t] = jnp.minimum(cm[t], dist)
                        d.append(dist)
                    while len(d) > 1:
                        d = [jnp.minimum(d[i], d[i + 1])
                             for i in range(0, len(d) - 1, 2)] + (
                                 [d[-1]] if len(d) % 2 else [])
                    rm = jnp.minimum(rm, d[0])
                rm = rot1(rm)
                for t in range(JV):
                    cmin_v[sls[t]] = cm[t]
                return rm

            rmin = lax.fori_loop(
                0, CHS // (L * JV), body_jq, jnp.full((L,), INF, jnp.float32)
            )
            rpart_v[gsl] = rmin
            return 0

        lax.fori_loop(0, N // L, body_ib, 0)

        pltpu.sync_copy(rpart_v, rowpart_hbm.at[b, ch])
        pltpu.sync_copy(cmin_v, colmin_hbm.at[b, psl])

    return body(go_t, gq_t, po_t, pq2_t)


def _tc_chamfer(gts, po_t, gq3, pq2t):
    def body(go_ref, po_ref, gq8_ref, pq2_ref, rp_ref, cm_ref):
        it = pl.program_id(1)
        zz2 = jnp.dot(gq8_ref[0], pq2_ref[0],
                      preferred_element_type=jnp.float32)
        po = po_ref[0]
        pn = po[0] * po[0] + po[1] * po[1] + po[2] * po[2]
        go = go_ref[0]
        gn = jnp.sum(go * go, axis=1)
        P = zz2 + (gn[:, None] + pn[None, :])
        rp_ref[0, 0, pl.ds(it * TI, TI)] = jnp.min(P, axis=1)
        ctile = jnp.min(P, axis=0)

        @pl.when(it == 0)
        def _():
            cm_ref[0, 0] = ctile

        @pl.when(it > 0)
        def _():
            cm_ref[0, 0] = jnp.minimum(cm_ref[0, 0], ctile)

    grid = (B, N // TI)
    rowpart, colmin = pl.pallas_call(
        body,
        grid=grid,
        in_specs=[
            pl.BlockSpec((1, TI, 3), lambda b, i: (b, i, 0)),
            pl.BlockSpec((1, 3, NPT), lambda b, i: (b, 0, 0)),
            pl.BlockSpec((1, TI, 3), lambda b, i: (b, i, 0)),
            pl.BlockSpec((1, 3, NPT), lambda b, i: (b, 0, 0)),
        ],
        out_specs=[
            pl.BlockSpec((1, 1, N), lambda b, i: (b, 0, 0)),
            pl.BlockSpec((1, 1, NPT), lambda b, i: (b, 0, 0)),
        ],
        out_shape=[
            jax.ShapeDtypeStruct((B, 1, N), jnp.float32),
            jax.ShapeDtypeStruct((B, 1, NPT), jnp.float32),
        ],
    )(gts, po_t, gq3, pq2t)
    return rowpart, colmin[:, 0, :]


def _combine_tc(rp_tc, rp_sc, cm_sc, cm_tc):
    def body(rpt_ref, rps_ref, cms_ref, cmt_ref, l1_ref, l2_ref):
        s1 = jnp.sum(cms_ref[...], axis=-1) + jnp.sum(cmt_ref[...], axis=-1)
        l1 = s1 * (1.0 / N)
        rp = jnp.minimum(jnp.min(rps_ref[...], axis=1), rpt_ref[:, 0, :])
        l2 = jnp.mean(rp, axis=-1)
        l1_ref[...] = jnp.broadcast_to(l1[:, None], (B, 128))
        l2_ref[...] = jnp.broadcast_to(l2[:, None], (B, 128))

    out = pl.pallas_call(
        body,
        out_shape=[
            jax.ShapeDtypeStruct((B, 128), jnp.float32),
            jax.ShapeDtypeStruct((B, 128), jnp.float32),
        ],
    )(rp_tc, rp_sc, cm_sc, cm_tc)
    return out[0][:, 0], out[1][:, 0]


def kernel(preds, gts):
    gq16, pq16 = lax.optimization_barrier(
        (gts.astype(jnp.bfloat16), preds.astype(jnp.bfloat16))
    )
    pq2 = (-2.0 * pq16.astype(jnp.float32)).astype(jnp.bfloat16)

    go_t = jnp.transpose(gts, (0, 2, 1))
    gq_t = jnp.transpose(gq16, (0, 2, 1)).astype(jnp.float32)
    po_sc = jnp.transpose(preds[:, :NPS, :], (0, 2, 1))
    pq2_sc = jnp.transpose(pq2[:, :NPS, :], (0, 2, 1)).astype(jnp.float32)
    rowpart_sc, colmin_sc = _sc_chamfer(go_t, gq_t, po_sc, pq2_sc)

    po_tc = jnp.transpose(preds[:, NPS:, :], (0, 2, 1))
    pq2t_tc = jnp.transpose(pq2[:, NPS:, :], (0, 2, 1))
    rowpart_tc, colmin_tc = _tc_chamfer(gts, po_tc, gq16, pq2t_tc)

    loss_1, loss_2 = _combine_tc(rowpart_tc, rowpart_sc, colmin_sc, colmin_tc)
    return (loss_1, loss_2)


# speedup vs baseline: 4.2708x; 1.1152x over previous
import functools

import jax
import jax.numpy as jnp
from jax import lax
from jax.experimental import pallas as pl
from jax.experimental.pallas import tpu as pltpu
from jax.experimental.pallas import tpu_sc as plsc

B = 8
N = 4096
NPS = 512
NPT = N - NPS
NCHUNK = 4
CHS = NPS // NCHUNK
L = 16
JV = 4
TI = 512
INF = float("inf")


def _sc_chamfer(go_t, gq_t, po_t, pq2_t):
    mesh = plsc.VectorSubcoreMesh(core_axis_name="c", subcore_axis_name="s")

    @functools.partial(
        pl.kernel,
        mesh=mesh,
        out_type=[
            jax.ShapeDtypeStruct((B, NCHUNK, N), jnp.float32),
            jax.ShapeDtypeStruct((B, NPS), jnp.float32),
        ],
        scratch_types=[
            pltpu.VMEM((3, N), jnp.float32),
            pltpu.VMEM((3, N), jnp.float32),
            pltpu.VMEM((3, CHS), jnp.float32),
            pltpu.VMEM((3, CHS), jnp.float32),
            pltpu.VMEM((N,), jnp.float32),
            pltpu.VMEM((CHS,), jnp.float32),
            pltpu.VMEM((CHS,), jnp.float32),
            pltpu.VMEM((N,), jnp.float32),
        ],
    )
    def body(go_hbm, gq_hbm, po_hbm, pq2_hbm, rowpart_hbm, colmin_hbm,
             go_v, gq_v, po_v, pq2_v, gn_v, pn_v, cmin_v, rpart_v):
        c = lax.axis_index("c")
        s = lax.axis_index("s")
        wid = c * 16 + s
        b = wid // NCHUNK
        ch = wid % NCHUNK

        pltpu.sync_copy(go_hbm.at[b], go_v)
        pltpu.sync_copy(gq_hbm.at[b], gq_v)
        psl = pl.ds(ch * CHS, CHS)
        pltpu.sync_copy(po_hbm.at[b, :, psl], po_v)
        pltpu.sync_copy(pq2_hbm.at[b, :, psl], pq2_v)

        ridx = (lax.iota(jnp.int32, L) + 1) & (L - 1)

        def rot1(v):
            return jnp.take_along_axis(v, ridx, axis=0)

        def init_gn(iv, _):
            sl = pl.ds(iv * L, L)
            x = go_v[0, sl]
            y = go_v[1, sl]
            z = go_v[2, sl]
            gn_v[sl] = x * x + y * y + z * z
            return 0

        lax.fori_loop(0, N // L, init_gn, 0)

        def init_pn(jv, _):
            sl = pl.ds(jv * L, L)
            x = po_v[0, sl]
            y = po_v[1, sl]
            z = po_v[2, sl]
            pn_v[sl] = x * x + y * y + z * z
            cmin_v[sl] = jnp.full((L,), INF, jnp.float32)
            return 0

        lax.fori_loop(0, CHS // L, init_pn, 0)

        def body_ib(ib, _):
            gsl = pl.ds(ib * L, L)
            gx0 = gq_v[0, gsl]
            gy0 = gq_v[1, gsl]
            gz0 = gq_v[2, gsl]
            gn0 = gn_v[gsl]

            def body_jq(jq, rmin):
                sls = [pl.ds((jq * JV + t) * L, L) for t in range(JV)]
                px = [pq2_v[0, sl] for sl in sls]
                py = [pq2_v[1, sl] for sl in sls]
                pz = [pq2_v[2, sl] for sl in sls]
                cm = [cmin_v[sl] for sl in sls]
                pn = [pn_v[sl] for sl in sls]
                gx, gy, gz, gn, rm = gx0, gy0, gz0, gn0, rmin
                for k in range(L):
                    if k > 0:
                        gx = rot1(gx)
                        gy = rot1(gy)
                        gz = rot1(gz)
                        gn = rot1(gn)
                        rm = rot1(rm)
                    d = []
                    for t in range(JV):
                        t3 = gx * px[t] + gy * py[t] + gz * pz[t]
                        dist = t3 + (gn + pn[t])
                        cm[t] = jnp.minimum(cm[t], dist)
                        d.append(dist)
                    while len(d) > 1:
                        d = [jnp.minimum(d[i], d[i + 1])
                             for i in range(0, len(d) - 1, 2)] + (
                                 [d[-1]] if len(d) % 2 else [])
                    rm = jnp.minimum(rm, d[0])
                rm = rot1(rm)
                for t in range(JV):
                    cmin_v[sls[t]] = cm[t]
                return rm

            rmin = lax.fori_loop(
                0, CHS // (L * JV), body_jq, jnp.full((L,), INF, jnp.float32)
            )
            rpart_v[gsl] = rmin
            return 0

        lax.fori_loop(0, N // L, body_ib, 0)

        pltpu.sync_copy(rpart_v, rowpart_hbm.at[b, ch])
        pltpu.sync_copy(cmin_v, colmin_hbm.at[b, psl])

    return body(go_t, gq_t, po_t, pq2_t)


def _tc_chamfer(lhs8, rhs8):
    def body(lhs_ref, rhs_ref, rp_ref, cm_ref):
        it = pl.program_id(1)
        P = jnp.dot(lhs_ref[0], rhs_ref[0],
                    preferred_element_type=jnp.float32)
        rp_ref[0, 0, pl.ds(it * TI, TI)] = jnp.min(P, axis=1)
        ctile = jnp.min(P, axis=0)

        @pl.when(it == 0)
        def _():
            cm_ref[0, 0] = ctile

        @pl.when(it > 0)
        def _():
            cm_ref[0, 0] = jnp.minimum(cm_ref[0, 0], ctile)

    grid = (B, N // TI)
    rowpart, colmin = pl.pallas_call(
        body,
        grid=grid,
        in_specs=[
            pl.BlockSpec((1, TI, 8), lambda b, i: (b, i, 0)),
            pl.BlockSpec((1, 8, NPT), lambda b, i: (b, 0, 0)),
        ],
        out_specs=[
            pl.BlockSpec((1, 1, N), lambda b, i: (b, 0, 0)),
            pl.BlockSpec((1, 1, NPT), lambda b, i: (b, 0, 0)),
        ],
        out_shape=[
            jax.ShapeDtypeStruct((B, 1, N), jnp.float32),
            jax.ShapeDtypeStruct((B, 1, NPT), jnp.float32),
        ],
    )(lhs8, rhs8)
    return rowpart, colmin[:, 0, :]


def _combine_tc(rp_tc, rp_sc, cm_sc, cm_tc):
    def body(rpt_ref, rps_ref, cms_ref, cmt_ref, l1_ref, l2_ref):
        s1 = jnp.sum(cms_ref[...], axis=-1) + jnp.sum(cmt_ref[...], axis=-1)
        l1 = s1 * (1.0 / N)
        rp = jnp.minimum(jnp.min(rps_ref[...], axis=1), rpt_ref[:, 0, :])
        l2 = jnp.mean(rp, axis=-1)
        l1_ref[...] = jnp.broadcast_to(l1[:, None], (B, 128))
        l2_ref[...] = jnp.broadcast_to(l2[:, None], (B, 128))

    out = pl.pallas_call(
        body,
        out_shape=[
            jax.ShapeDtypeStruct((B, 128), jnp.float32),
            jax.ShapeDtypeStruct((B, 128), jnp.float32),
        ],
    )(rp_tc, rp_sc, cm_sc, cm_tc)
    return out[0][:, 0], out[1][:, 0]


def kernel(preds, gts):
    gq16, pq16 = lax.optimization_barrier(
        (gts.astype(jnp.bfloat16), preds.astype(jnp.bfloat16))
    )
    pq2 = (-2.0 * pq16.astype(jnp.float32)).astype(jnp.bfloat16)

    go_t = jnp.transpose(gts, (0, 2, 1))
    gq_t = jnp.transpose(gq16, (0, 2, 1)).astype(jnp.float32)
    po_sc = jnp.transpose(preds[:, :NPS, :], (0, 2, 1))
    pq2_sc = jnp.transpose(pq2[:, :NPS, :], (0, 2, 1)).astype(jnp.float32)
    rowpart_sc, colmin_sc = _sc_chamfer(go_t, gq_t, po_sc, pq2_sc)

    gn = jnp.sum(gts * gts, axis=-1)
    gnh = gn.astype(jnp.bfloat16)
    gnl = (gn - gnh.astype(jnp.float32)).astype(jnp.bfloat16)
    pn = jnp.sum(preds * preds, axis=-1)[:, NPS:]
    pnh = pn.astype(jnp.bfloat16)
    pnl = (pn - pnh.astype(jnp.float32)).astype(jnp.bfloat16)
    one_g = jnp.ones((B, N), jnp.bfloat16)
    zero_g = jnp.zeros((B, N), jnp.bfloat16)
    lhs8 = jnp.stack(
        [gq16[..., 0], gq16[..., 1], gq16[..., 2], gnh, gnl,
         one_g, one_g, zero_g], axis=-1)
    one_p = jnp.ones((B, NPT), jnp.bfloat16)
    zero_p = jnp.zeros((B, NPT), jnp.bfloat16)
    pq2_tc = pq2[:, NPS:, :]
    rhs8 = jnp.stack(
        [pq2_tc[..., 0], pq2_tc[..., 1], pq2_tc[..., 2],
         one_p, one_p, pnh, pnl, zero_p], axis=1)
    rowpart_tc, colmin_tc = _tc_chamfer(lhs8, rhs8)

    loss_1, loss_2 = _combine_tc(rowpart_tc, rowpart_sc, colmin_sc, colmin_tc)
    return (loss_1, loss_2)
